# initial kernel scaffold (unmeasured)
import jax
import jax.numpy as jnp
from jax import lax
from jax.experimental import pallas as pl
from jax.experimental.pallas import tpu as pltpu

N_DEV = 16


def kernel(x, w_mat, scale_x, scale_w):
    m_per, k = x.shape
    _, n = w_mat.shape
    n_per = n // N_DEV

    def body(x_ref, w_ref, sx_ref, sw_ref, out_ref, y_ref, send_sems, recv_sems):
        me = lax.axis_index("i")

        barrier = pltpu.get_barrier_semaphore()
        for d in range(1, N_DEV):
            p = (me + d) % N_DEV
            pl.semaphore_signal(
                barrier, inc=1, device_id=(p,),
                device_id_type=pl.DeviceIdType.MESH,
            )
        pl.semaphore_wait(barrier, N_DEV - 1)

        s = sx_ref[0] * sw_ref[0]
        xf = x_ref[...]

        sends = []
        for d in range(N_DEV):
            j = (me + d) % N_DEV
            yj = jnp.dot(
                xf,
                pl.load(w_ref, (slice(None), pl.ds(j * n_per, n_per))),
                preferred_element_type=jnp.float32,
            )
            yj = yj * s
            yj = yj * (1.0 / (1.0 + jnp.exp(-jnp.clip(yj, -60.0, 60.0))))

            if d == 0:
                @pl.when(True)
                def _():
                    pass
                out_ref[pl.ds(me * m_per, m_per), :] = yj
            else:
                pl.store(
                    y_ref,
                    (pl.ds(j, 1), slice(None), slice(None)),
                    yj[None],
                )
                rdma = pltpu.make_async_remote_copy(
                    src_ref=y_ref.at[j],
                    dst_ref=out_ref.at[pl.ds(me * m_per, m_per), :],
                    send_sem=send_sems.at[d],
                    recv_sem=recv_sems.at[me],
                    device_id=(j,),
                    device_id_type=pl.DeviceIdType.MESH,
                )
                rdma.start()
                sends.append(rdma)

        for d in range(1, N_DEV):
            p = (me + d) % N_DEV
            recv = pltpu.make_async_remote_copy(
                src_ref=y_ref.at[p],
                dst_ref=out_ref.at[pl.ds(p * m_per, m_per), :],
                send_sem=send_sems.at[d],
                recv_sem=recv_sems.at[p],
                device_id=(p,),
                device_id_type=pl.DeviceIdType.MESH,
            )
            recv.wait_recv()

        for rdma in sends:
            rdma.wait_send()

    return pl.pallas_call(
        body,
        out_shape=jax.ShapeDtypeStruct((N_DEV * m_per, n_per), jnp.float32),
        in_specs=[
            pl.BlockSpec(memory_space=pltpu.VMEM),
            pl.BlockSpec(memory_space=pltpu.VMEM),
            pl.BlockSpec(memory_space=pltpu.SMEM),
            pl.BlockSpec(memory_space=pltpu.SMEM),
        ],
        out_specs=pl.BlockSpec(memory_space=pltpu.VMEM),
        scratch_shapes=[
            pltpu.VMEM((N_DEV, m_per, n_per), jnp.float32),
            pltpu.SemaphoreType.DMA((N_DEV,)),
            pltpu.SemaphoreType.DMA((N_DEV,)),
        ],
        compiler_params=pltpu.CompilerParams(collective_id=0),
    )(x, w_mat, scale_x, scale_w)


# baseline (device time: 103973 ns/iter reference)
import jax
import jax.numpy as jnp
from jax import lax
from jax.experimental import pallas as pl
from jax.experimental.pallas import tpu as pltpu

N_DEV = 16


def kernel(x, w_mat, scale_x, scale_w):
    m_per, k = x.shape
    _, n = w_mat.shape
    n_per = n // N_DEV

    def body(x_ref, w_ref, sx_ref, sw_ref, out_ref,
             wbuf, y_ref, load_sems, send_sems, recv_sems):
        me = lax.axis_index("i")

        barrier = pltpu.get_barrier_semaphore()
        for d in range(1, N_DEV):
            p = (me + d) % N_DEV
            pl.semaphore_signal(
                barrier, inc=1, device_id=(p,),
                device_id_type=pl.DeviceIdType.MESH,
            )
        pl.semaphore_wait(barrier, N_DEV - 1)

        def start_load(slot, j):
            cp = pltpu.make_async_copy(
                w_ref.at[:, pl.ds(j * n_per, n_per)],
                wbuf.at[slot],
                load_sems.at[slot],
            )
            cp.start()
            return cp

        s = sx_ref[0] * sw_ref[0]
        xb = x_ref[...].astype(jnp.bfloat16)

        loads = [None, None]
        loads[0] = start_load(0, me)

        sends = []
        for d in range(N_DEV):
            j = (me + d) % N_DEV
            slot = d % 2
            if d + 1 < N_DEV:
                loads[(d + 1) % 2] = start_load((d + 1) % 2, (me + d + 1) % N_DEV)
            loads[slot].wait()
            yj = jnp.dot(
                xb,
                wbuf[slot].astype(jnp.bfloat16),
                preferred_element_type=jnp.float32,
            )
            yj = yj * s
            yj = yj * (1.0 / (1.0 + jnp.exp(-jnp.clip(yj, -60.0, 60.0))))

            if d == 0:
                out_ref[pl.ds(me * m_per, m_per), :] = yj
            else:
                y_ref[pl.ds(j, 1), :, :] = yj[None]
                rdma = pltpu.make_async_remote_copy(
                    src_ref=y_ref.at[j],
                    dst_ref=out_ref.at[pl.ds(me * m_per, m_per), :],
                    send_sem=send_sems.at[d],
                    recv_sem=recv_sems.at[me],
                    device_id=(j,),
                    device_id_type=pl.DeviceIdType.MESH,
                )
                rdma.start()
                sends.append(rdma)

        for d in range(1, N_DEV):
            p = (me + d) % N_DEV
            recv = pltpu.make_async_remote_copy(
                src_ref=y_ref.at[p],
                dst_ref=out_ref.at[pl.ds(p * m_per, m_per), :],
                send_sem=send_sems.at[d],
                recv_sem=recv_sems.at[p],
                device_id=(p,),
                device_id_type=pl.DeviceIdType.MESH,
            )
            recv.wait_recv()

        for rdma in sends:
            rdma.wait_send()

    return pl.pallas_call(
        body,
        out_shape=jax.ShapeDtypeStruct((N_DEV * m_per, n_per), jnp.float32),
        in_specs=[
            pl.BlockSpec(memory_space=pltpu.VMEM),
            pl.BlockSpec(memory_space=pl.ANY),
            pl.BlockSpec(memory_space=pltpu.SMEM),
            pl.BlockSpec(memory_space=pltpu.SMEM),
        ],
        out_specs=pl.BlockSpec(memory_space=pltpu.VMEM),
        scratch_shapes=[
            pltpu.VMEM((2, k, n_per), w_mat.dtype),
            pltpu.VMEM((N_DEV, m_per, n_per), jnp.float32),
            pltpu.SemaphoreType.DMA((2,)),
            pltpu.SemaphoreType.DMA((N_DEV,)),
            pltpu.SemaphoreType.DMA((N_DEV,)),
        ],
        compiler_params=pltpu.CompilerParams(collective_id=0),
    )(x, w_mat, scale_x, scale_w)


# device time: 71248 ns/iter; 1.4593x vs baseline; 1.4593x over previous
import jax
import jax.numpy as jnp
from jax import lax
from jax.experimental import pallas as pl
from jax.experimental.pallas import tpu as pltpu

N_DEV = 16


def kernel(x, w_mat, scale_x, scale_w):
    m_per, k = x.shape
    _, n = w_mat.shape
    n_per = n // N_DEV

    def body(x_ref, w_ref, sx_ref, sw_ref, out_ref,
             wbuf, y_ref, rbuf, load_sems, send_sems, recv_sems):
        me = lax.axis_index("i")

        barrier = pltpu.get_barrier_semaphore()
        for d in range(1, N_DEV):
            p = (me + d) % N_DEV
            pl.semaphore_signal(
                barrier, inc=1, device_id=(p,),
                device_id_type=pl.DeviceIdType.MESH,
            )
        pl.semaphore_wait(barrier, N_DEV - 1)

        def start_load(slot, j):
            cp = pltpu.make_async_copy(
                w_ref.at[:, pl.ds(j * n_per, n_per)],
                wbuf.at[slot],
                load_sems.at[slot],
            )
            cp.start()
            return cp

        s = sx_ref[0] * sw_ref[0]
        xb = x_ref[...].astype(jnp.bfloat16)

        loads = [None, None]
        loads[0] = start_load(0, me)

        sends = []
        for d in range(N_DEV):
            j = (me + d) % N_DEV
            slot = d % 2
            if d + 1 < N_DEV:
                loads[(d + 1) % 2] = start_load((d + 1) % 2, (me + d + 1) % N_DEV)
            loads[slot].wait()
            yj = jnp.dot(
                xb,
                wbuf[slot].astype(jnp.bfloat16),
                preferred_element_type=jnp.float32,
            )
            yj = yj * s
            yj = yj * (1.0 / (1.0 + jnp.exp(-jnp.clip(yj, -60.0, 60.0))))

            if d == 0:
                out_ref[pl.ds(me * m_per, m_per), :] = yj
            else:
                y_ref[pl.ds(j, 1), :, :] = yj.astype(jnp.bfloat16)[None]
                rdma = pltpu.make_async_remote_copy(
                    src_ref=y_ref.at[j],
                    dst_ref=rbuf.at[me],
                    send_sem=send_sems.at[d],
                    recv_sem=recv_sems.at[me],
                    device_id=(j,),
                    device_id_type=pl.DeviceIdType.MESH,
                )
                rdma.start()
                sends.append(rdma)

        for d in range(1, N_DEV):
            p = (me + d) % N_DEV
            recv = pltpu.make_async_remote_copy(
                src_ref=y_ref.at[p],
                dst_ref=rbuf.at[p],
                send_sem=send_sems.at[d],
                recv_sem=recv_sems.at[p],
                device_id=(p,),
                device_id_type=pl.DeviceIdType.MESH,
            )
            recv.wait_recv()
            out_ref[pl.ds(p * m_per, m_per), :] = rbuf[p].astype(jnp.float32)

        for rdma in sends:
            rdma.wait_send()

    return pl.pallas_call(
        body,
        out_shape=jax.ShapeDtypeStruct((N_DEV * m_per, n_per), jnp.float32),
        in_specs=[
            pl.BlockSpec(memory_space=pltpu.VMEM),
            pl.BlockSpec(memory_space=pl.ANY),
            pl.BlockSpec(memory_space=pltpu.SMEM),
            pl.BlockSpec(memory_space=pltpu.SMEM),
        ],
        out_specs=pl.BlockSpec(memory_space=pltpu.VMEM),
        scratch_shapes=[
            pltpu.VMEM((2, k, n_per), w_mat.dtype),
            pltpu.VMEM((N_DEV, m_per, n_per), jnp.bfloat16),
            pltpu.VMEM((N_DEV, m_per, n_per), jnp.bfloat16),
            pltpu.SemaphoreType.DMA((2,)),
            pltpu.SemaphoreType.DMA((N_DEV,)),
            pltpu.SemaphoreType.DMA((N_DEV,)),
        ],
        compiler_params=pltpu.CompilerParams(collective_id=0),
    )(x, w_mat, scale_x, scale_w)


# device time: 50122 ns/iter; 2.0744x vs baseline; 1.4215x over previous
import os

import jax
import jax.numpy as jnp
from jax import lax
from jax.experimental import pallas as pl
from jax.experimental.pallas import tpu as pltpu

N_DEV = 16

_VARIANT = os.environ.get("KVARIANT", "full")
_DO_COMPUTE = _VARIANT in ("full", "compute_only")
_DO_COMM = _VARIANT in ("full", "comm_only")


def kernel(x, w_mat, scale_x, scale_w):
    m_per, k = x.shape
    _, n = w_mat.shape
    n_per = n // N_DEV

    def body(x_ref, w_ref, sx_ref, sw_ref, out_ref,
             wbuf, y_ref, rbuf, load_sems, send_sems, recv_sems):
        me = lax.axis_index("i")

        if _DO_COMM:
            barrier = pltpu.get_barrier_semaphore()
            for d in range(1, N_DEV):
                p = (me + d) % N_DEV
                pl.semaphore_signal(
                    barrier, inc=1, device_id=(p,),
                    device_id_type=pl.DeviceIdType.MESH,
                )
            pl.semaphore_wait(barrier, N_DEV - 1)

        def start_load(slot, j):
            cp = pltpu.make_async_copy(
                w_ref.at[:, pl.ds(j * n_per, n_per)],
                wbuf.at[slot],
                load_sems.at[slot],
            )
            cp.start()
            return cp

        s = sx_ref[0] * sw_ref[0]
        xb = x_ref[...].astype(jnp.bfloat16)

        loads = [None, None]
        if _DO_COMPUTE:
            loads[0] = start_load(0, me)

        sends = []
        for d in range(N_DEV):
            j = (me + d) % N_DEV
            slot = d % 2
            if _DO_COMPUTE:
                if d + 1 < N_DEV:
                    loads[(d + 1) % 2] = start_load(
                        (d + 1) % 2, (me + d + 1) % N_DEV
                    )
                loads[slot].wait()
                yj = jnp.dot(
                    xb,
                    wbuf[slot].astype(jnp.bfloat16),
                    preferred_element_type=jnp.float32,
                )
                yj = yj * s
                yj = yj * (1.0 / (1.0 + jnp.exp(-jnp.clip(yj, -60.0, 60.0))))
            else:
                yj = jnp.zeros((m_per, n_per), jnp.float32)

            if d == 0:
                out_ref[pl.ds(me * m_per, m_per), :] = yj
            elif _DO_COMM:
                y_ref[pl.ds(j, 1), :, :] = yj.astype(jnp.bfloat16)[None]
                rdma = pltpu.make_async_remote_copy(
                    src_ref=y_ref.at[j],
                    dst_ref=rbuf.at[me],
                    send_sem=send_sems.at[d],
                    recv_sem=recv_sems.at[me],
                    device_id=(j,),
                    device_id_type=pl.DeviceIdType.MESH,
                )
                rdma.start()
                sends.append(rdma)
            else:
                y_ref[pl.ds(j, 1), :, :] = yj.astype(jnp.bfloat16)[None]

        for d in range(1, N_DEV):
            p = (me + d) % N_DEV
            if _DO_COMM:
                recv = pltpu.make_async_remote_copy(
                    src_ref=y_ref.at[p],
                    dst_ref=rbuf.at[p],
                    send_sem=send_sems.at[d],
                    recv_sem=recv_sems.at[p],
                    device_id=(p,),
                    device_id_type=pl.DeviceIdType.MESH,
                )
                recv.wait_recv()
            out_ref[pl.ds(p * m_per, m_per), :] = rbuf[p].astype(jnp.float32)

        for rdma in sends:
            rdma.wait_send()

    return pl.pallas_call(
        body,
        out_shape=jax.ShapeDtypeStruct((N_DEV * m_per, n_per), jnp.float32),
        in_specs=[
            pl.BlockSpec(memory_space=pltpu.VMEM),
            pl.BlockSpec(memory_space=pl.ANY),
            pl.BlockSpec(memory_space=pltpu.SMEM),
            pl.BlockSpec(memory_space=pltpu.SMEM),
        ],
        out_specs=pl.BlockSpec(memory_space=pltpu.VMEM),
        scratch_shapes=[
            pltpu.VMEM((2, k, n_per), w_mat.dtype),
            pltpu.VMEM((N_DEV, m_per, n_per), jnp.bfloat16),
            pltpu.VMEM((N_DEV, m_per, n_per), jnp.bfloat16),
            pltpu.SemaphoreType.DMA((2,)),
            pltpu.SemaphoreType.DMA((N_DEV,)),
            pltpu.SemaphoreType.DMA((N_DEV,)),
        ],
        compiler_params=pltpu.CompilerParams(collective_id=0),
    )(x, w_mat, scale_x, scale_w)


# device time: 49395 ns/iter; 2.1049x vs baseline; 1.0147x over previous
import os

import jax
import jax.numpy as jnp
from jax import lax
from jax.experimental import pallas as pl
from jax.experimental.pallas import tpu as pltpu

N_DEV = 16

_VARIANT = os.environ.get("KVARIANT", "full")
_DO_COMPUTE = _VARIANT in ("full", "compute_only")
_DO_COMM = _VARIANT in ("full", "comm_only")


def kernel(x, w_mat, scale_x, scale_w):
    m_per, k = x.shape
    _, n = w_mat.shape
    n_per = n // N_DEV

    def body(x_ref, w_ref, sx_ref, sw_ref, out_ref,
             wbuf, y_ref, rbuf, load_sems, send_sems, recv_sems):
        me = lax.axis_index("i")

        if _DO_COMM:
            barrier = pltpu.get_barrier_semaphore()
            for d in range(1, N_DEV):
                p = (me + d) % N_DEV
                pl.semaphore_signal(
                    barrier, inc=1, device_id=(p,),
                    device_id_type=pl.DeviceIdType.MESH,
                )
            pl.semaphore_wait(barrier, N_DEV - 1)

        def start_load(slot, j):
            cp = pltpu.make_async_copy(
                w_ref.at[:, pl.ds(j * n_per, n_per)],
                wbuf.at[slot],
                load_sems.at[slot],
            )
            cp.start()
            return cp

        s = sx_ref[0] * sw_ref[0]
        xb = x_ref[...].astype(jnp.bfloat16)

        loads = [None, None]
        if _DO_COMPUTE:
            loads[0] = start_load(0, me)

        sends = []
        for d in range(N_DEV):
            j = (me + d) % N_DEV
            slot = d % 2
            if _DO_COMPUTE:
                if d + 1 < N_DEV:
                    loads[(d + 1) % 2] = start_load(
                        (d + 1) % 2, (me + d + 1) % N_DEV
                    )
                loads[slot].wait()
                yj = jnp.dot(
                    xb,
                    wbuf[slot].astype(jnp.bfloat16),
                    preferred_element_type=jnp.float32,
                )
                yj = yj * s
                yj = yj * (1.0 / (1.0 + jnp.exp(-jnp.clip(yj, -60.0, 60.0))))
            else:
                yj = jnp.zeros((m_per, n_per), jnp.float32)

            if d == 0:
                out_ref[pl.ds(me * m_per, m_per), :] = yj
            elif _DO_COMM:
                y_ref[pl.ds(j, 1), :, :] = yj.astype(jnp.bfloat16)[None]
                rdma = pltpu.make_async_remote_copy(
                    src_ref=y_ref.at[j],
                    dst_ref=rbuf.at[me],
                    send_sem=send_sems.at[d],
                    recv_sem=recv_sems.at[me],
                    device_id=(j,),
                    device_id_type=pl.DeviceIdType.MESH,
                )
                rdma.start()
                sends.append(rdma)
            else:
                y_ref[pl.ds(j, 1), :, :] = yj.astype(jnp.bfloat16)[None]

        for d in range(1, N_DEV):
            p = (me + d) % N_DEV
            if _DO_COMM:
                recv = pltpu.make_async_remote_copy(
                    src_ref=y_ref.at[p],
                    dst_ref=rbuf.at[p],
                    send_sem=send_sems.at[d],
                    recv_sem=recv_sems.at[p],
                    device_id=(p,),
                    device_id_type=pl.DeviceIdType.MESH,
                )
                recv.wait_recv()
            out_ref[pl.ds(p * m_per, m_per), :] = rbuf[p].astype(jnp.float32)

        for rdma in sends:
            rdma.wait_send()

    return pl.pallas_call(
        body,
        out_shape=jax.ShapeDtypeStruct((N_DEV * m_per, n_per), jnp.float32),
        in_specs=[
            pl.BlockSpec(memory_space=pltpu.VMEM),
            pl.BlockSpec(memory_space=pl.ANY),
            pl.BlockSpec(memory_space=pltpu.SMEM),
            pl.BlockSpec(memory_space=pltpu.SMEM),
        ],
        out_specs=pl.BlockSpec(memory_space=pltpu.VMEM),
        scratch_shapes=[
            pltpu.VMEM((2, k, n_per), w_mat.dtype),
            pltpu.VMEM((N_DEV, m_per, n_per), jnp.bfloat16),
            pltpu.VMEM((N_DEV, m_per, n_per), jnp.bfloat16),
            pltpu.SemaphoreType.DMA((2,)),
            pltpu.SemaphoreType.DMA((N_DEV,)),
            pltpu.SemaphoreType.DMA((N_DEV,)),
        ],
        compiler_params=pltpu.CompilerParams(
            collective_id=0 if _DO_COMM else None
        ),
    )(x, w_mat, scale_x, scale_w)
